# baseline (speedup 1.0000x reference)
import math

import numpy as np
import jax
import jax.numpy as jnp
from jax.experimental import pallas as pl

_B = 128
_L = 512
_C = 14
_D = 512
_TOPK = 5
_LAMBDA = 0.01
_FREQ = 256
_HID = _FREQ * 4

_PS = [int(p) for p in
       np.unique(np.floor(1.0 / np.fft.rfftfreq(_L)[1:]).astype(np.int64))]
_NE = len(_PS)
_EP = 48
_LPS = [int(math.ceil(_L / p)) for p in _PS]
_FLAT = [lp * p * _C for lp, p in zip(_LPS, _PS)]
_TMAX = max(lp * p for lp, p in zip(_LPS, _PS))
_EPS = np.float32(np.finfo(float).eps)
_RBR = 224
_RB = 8

_n = np.arange(_L, dtype=np.float64)[:, None]
_k = np.arange(1, _FREQ + 1, dtype=np.float64)[None, :]
_ANG = 2.0 * np.pi * _n * _k / _L
_CM = np.cos(_ANG).astype(np.float32)
_SM = (-np.sin(_ANG)).astype(np.float32)

_SEL = np.zeros((16, _RBR), dtype=np.float32)
for _m in range(16):
    _SEL[_m, _m * _C:(_m + 1) * _C] = 1.0 / _C


def _pos_embed(Lp):
    position = np.arange(Lp, dtype=np.float32)[:, None]
    div_term = np.exp(np.arange(0, _D, 2, dtype=np.float32)
                      * -(math.log(10000.0) / _D))
    pe = np.zeros((Lp, _D), dtype=np.float32)
    pe[:, 0::2] = np.sin(position * div_term)
    pe[:, 1::2] = np.cos(position * div_term)
    return pe

_PES = [_pos_embed(lp) for lp in _LPS]

_HI = jax.lax.Precision.HIGHEST


def _softshrink(v):
    return jnp.where(v > _LAMBDA, v - _LAMBDA,
                     jnp.where(v < -_LAMBDA, v + _LAMBDA, 0.0))


def _router_body(xr_ref, xi_ref, w1r_ref, w1i_ref, b1r_ref, b1i_ref,
                 w2r_ref, w2i_ref, b2r_ref, b2i_ref, sel_ref, gates_ref):
    bf = jnp.bfloat16
    f32 = jnp.float32
    xr = xr_ref[...].astype(bf)
    xi = xi_ref[...].astype(bf)
    w1r = w1r_ref[...].astype(bf)
    w1i = w1i_ref[...].astype(bf)
    o1r = jnp.maximum(
        jax.lax.dot(xr, w1r, preferred_element_type=f32)
        - jax.lax.dot(xi, w1i, preferred_element_type=f32)
        + b1r_ref[...], 0.0)
    o1i = jnp.maximum(
        jax.lax.dot(xi, w1r, preferred_element_type=f32)
        + jax.lax.dot(xr, w1i, preferred_element_type=f32)
        + b1i_ref[...], 0.0)
    o1rb = o1r.astype(bf)
    o1ib = o1i.astype(bf)
    w2r = w2r_ref[...].astype(bf)
    w2i = w2i_ref[...].astype(bf)
    o2r = (jax.lax.dot(o1rb, w2r, preferred_element_type=f32)
           - jax.lax.dot(o1ib, w2i, preferred_element_type=f32)
           + b2r_ref[...])
    o2i = (jax.lax.dot(o1ib, w2r, preferred_element_type=f32)
           + jax.lax.dot(o1rb, w2i, preferred_element_type=f32)
           + b2i_ref[...])
    o2r = _softshrink(o2r)
    o2i = _softshrink(o2i)
    mag = jnp.sqrt(o2r * o2r + o2i * o2i)
    wts = jax.lax.dot(sel_ref[...], mag, precision=_HI)
    iota = jax.lax.broadcasted_iota(jnp.int32, (16, _EP), 1)
    work = jnp.where(iota < _NE, wts, -jnp.inf)
    vals, picks = [], []
    for _ in range(_TOPK):
        mx = jnp.max(work, axis=1, keepdims=True)
        cand = jnp.where(work == mx, iota, 2 * _EP)
        mn = jnp.min(cand, axis=1, keepdims=True)
        pick = iota == mn
        vals.append(mx)
        picks.append(pick)
        work = jnp.where(pick, -jnp.inf, work)
    vs = jnp.concatenate(vals, axis=1)
    vm = jnp.max(vs, axis=1, keepdims=True)
    ev = jnp.exp(vs - vm)
    g = ev / jnp.sum(ev, axis=1, keepdims=True)
    gates = jnp.zeros((16, _EP), jnp.float32)
    for kk in range(_TOPK):
        gates = gates + picks[kk].astype(jnp.float32) * g[:, kk:kk + 1]
    gates_ref[...] = gates


def _router(xr, xi, w1, b1, w2, b2):
    w2p = jnp.pad(w2, ((0, 0), (0, 0), (0, _EP - _NE)))
    b2p = jnp.pad(b2, ((0, 0), (0, _EP - _NE)))
    const = lambda s: (0, 0)
    return pl.pallas_call(
        _router_body,
        grid=(_B * _C // _RBR,),
        in_specs=[
            pl.BlockSpec((_RBR, _FREQ), lambda s: (s, 0)),
            pl.BlockSpec((_RBR, _FREQ), lambda s: (s, 0)),
            pl.BlockSpec((_FREQ, _HID), const),
            pl.BlockSpec((_FREQ, _HID), const),
            pl.BlockSpec((1, _HID), const),
            pl.BlockSpec((1, _HID), const),
            pl.BlockSpec((_HID, _EP), const),
            pl.BlockSpec((_HID, _EP), const),
            pl.BlockSpec((1, _EP), const),
            pl.BlockSpec((1, _EP), const),
            pl.BlockSpec((16, _RBR), const),
        ],
        out_specs=pl.BlockSpec((16, _EP), lambda s: (s, 0)),
        out_shape=jax.ShapeDtypeStruct((_B, _EP), jnp.float32),
    )(xr, xi,
      w1[0], w1[1], b1[0].reshape(1, -1), b1[1].reshape(1, -1),
      w2p[0], w2p[1], b2p[0].reshape(1, -1), b2p[1].reshape(1, -1),
      jnp.asarray(_SEL))


_UT = np.triu(np.ones((_B, _B), dtype=np.float32))
_ECH = 8


def _compact_body(gT_ref, ut_ref, rows_ref, gv_ref, cnt_ref):
    gT = gT_ref[...]
    m = (gT > 0.0).astype(jnp.float32)
    pos = jax.lax.dot(m, ut_ref[...], precision=_HI) - 1.0
    posi = pos.astype(jnp.int32)
    j3 = jax.lax.broadcasted_iota(jnp.int32, (_ECH, _B, _B), 2)
    oh = jnp.where((posi[:, :, None] == j3) & (m[:, :, None] > 0.0), 1.0, 0.0)
    rvec = jax.lax.broadcasted_iota(jnp.int32, (_ECH, _B, _B), 1).astype(jnp.float32)
    rows = jnp.sum(oh * rvec, axis=1)
    gv = jnp.sum(oh * gT[:, :, None], axis=1)
    cnt = jnp.sum(m, axis=1, keepdims=True)
    rows_ref[...] = rows.astype(jnp.int32)
    gv_ref[...] = gv
    cnt_ref[...] = jnp.broadcast_to(cnt, (_ECH, _B)).astype(jnp.int32)


def _compact(gates):
    gT = gates.T
    blk = lambda s: (s, 0)
    const = lambda s: (0, 0)
    return pl.pallas_call(
        _compact_body,
        grid=(_EP // _ECH,),
        in_specs=[
            pl.BlockSpec((_ECH, _B), blk),
            pl.BlockSpec((_B, _B), const),
        ],
        out_specs=[
            pl.BlockSpec((_ECH, _B), blk),
            pl.BlockSpec((_ECH, _B), blk),
            pl.BlockSpec((_ECH, _B), blk),
        ],
        out_shape=[
            jax.ShapeDtypeStruct((_EP, _B), jnp.int32),
            jax.ShapeDtypeStruct((_EP, _B), jnp.float32),
            jax.ShapeDtypeStruct((_EP, _B), jnp.int32),
        ],
    )(gT, jnp.asarray(_UT))


def _make_expert_body(i):
    Lp, K = _LPS[i], _PS[i] * _C

    def body(rows_ref, cnt_ref, gvT_ref, xp_ref, v_ref, w_ref, b_ref,
             pe_ref, o_ref):
        s = pl.program_id(0)

        @pl.when(s * _RB < cnt_ref[i, 0])
        def _():
            parts = [xp_ref[pl.ds(rows_ref[i, s * _RB + r], 1)]
                     for r in range(_RB)]
            xg = jnp.concatenate(parts, axis=0)
            xp = xg.reshape(_RB * Lp, K).astype(jnp.bfloat16)
            emb = jax.lax.dot(xp, v_ref[...].astype(jnp.bfloat16),
                              preferred_element_type=jnp.float32)
            pe = jnp.broadcast_to(pe_ref[...][None], (_RB, Lp, _D))
            emb = emb + pe.reshape(_RB * Lp, _D)
            y = jax.lax.dot(emb.astype(jnp.bfloat16),
                            w_ref[...].astype(jnp.bfloat16),
                            preferred_element_type=jnp.float32) + b_ref[...]
            gcol = gvT_ref[...][:, i:i + 1]
            o_ref[...] = jnp.exp(y.reshape(_RB, Lp, K)) * gcol[:, :, None]
    return body


def _expert(i, rows, cnts, gvT, Xp, v, w, b, pe):
    Lp, K = _LPS[i], _PS[i] * _C
    const = lambda s, rows, cnt: (0, 0)

    def omap(s, rows_sm, cnt_sm):
        nb = jnp.maximum((cnt_sm[i, 0] + _RB - 1) // _RB - 1, 0)
        return (jnp.minimum(s, nb), 0, 0)

    from jax.experimental.pallas import tpu as pltpu
    grid_spec = pltpu.PrefetchScalarGridSpec(
        num_scalar_prefetch=2,
        grid=(_B // _RB,),
        in_specs=[
            pl.BlockSpec((_RB, _EP), lambda s, rows, cnt: (s, 0)),
            pl.BlockSpec((_B, Lp, K), lambda s, rows, cnt: (0, 0, 0)),
            pl.BlockSpec((K, _D), const),
            pl.BlockSpec((_D, K), const),
            pl.BlockSpec((1, K), const),
            pl.BlockSpec((Lp, _D), const),
        ],
        out_specs=pl.BlockSpec((_RB, Lp, K), omap),
    )
    return pl.pallas_call(
        _make_expert_body(i),
        grid_spec=grid_spec,
        out_shape=jax.ShapeDtypeStruct((_B, Lp, K), jnp.float32),
    )(rows, cnts, gvT, Xp, v, w, b, pe)


def _combine_body(rows_ref, cnt_ref, *rest):
    yrefs = rest[:_NE]
    o_ref = rest[_NE]
    acc_ref = rest[_NE + 1]
    i = pl.program_id(0)
    s = pl.program_id(1)

    @pl.when((i == 0) & (s == 0))
    def _():
        acc_ref[...] = jnp.zeros((_B, _L * _C), jnp.float32)

    cnt = cnt_ref[i, 0]

    @pl.when(s * _RB < cnt)
    def _():
        yv = jax.lax.switch(
            i, [lambda j=j: yrefs[j][...][:, :_L * _C] for j in range(_NE)])
        rr = jax.lax.broadcasted_iota(jnp.int32, (_RB, 1), 0)
        valid = (s * _RB + rr) < cnt
        contrib = jnp.where(valid, yv, 0.0)
        for r in range(_RB):
            row = rows_ref[i, s * _RB + r]
            acc_ref[pl.ds(row, 1), :] += contrib[r:r + 1, :]

    @pl.when((i == _NE - 1) & (s == (_B // _RB) - 1))
    def _():
        a = acc_ref[...]
        o_ref[...] = jnp.log(jnp.where(a == 0.0, _EPS, a))


def _combine(rows, cnts, ys_flat):
    from jax.experimental.pallas import tpu as pltpu

    in_specs = []
    for j in range(_NE):
        def ymap(i, s, rows_sm, cnt_sm, j=j):
            nbj = jnp.maximum((cnt_sm[j, 0] + _RB - 1) // _RB - 1, 0)
            return (jnp.where(i < j, 0,
                              jnp.where(i > j, nbj, jnp.minimum(s, nbj))), 0)
        in_specs.append(pl.BlockSpec((_RB, _FLAT[j]), ymap))

    grid_spec = pltpu.PrefetchScalarGridSpec(
        num_scalar_prefetch=2,
        grid=(_NE, _B // _RB),
        in_specs=in_specs,
        out_specs=pl.BlockSpec((_B, _L * _C), lambda i, s, r, c: (0, 0)),
        scratch_shapes=[pltpu.VMEM((_B, _L * _C), jnp.float32)],
    )
    return pl.pallas_call(
        _combine_body,
        grid_spec=grid_spec,
        out_shape=jax.ShapeDtypeStruct((_B, _L * _C), jnp.float32),
    )(rows, cnts, *ys_flat)


def kernel(x, afno_params, ve_ws, head_ws, head_bs):
    w1, b1, w2, b2 = afno_params
    xf = jnp.fft.rfft(jnp.transpose(x, (0, 2, 1)), axis=-1)
    xr = jnp.real(xf)[:, :, 1:].reshape(_B * _C, _FREQ)
    xi = jnp.imag(xf)[:, :, 1:].reshape(_B * _C, _FREQ)
    gates = _router(xr, xi, w1, b1, w2, b2)
    rows, gv, cnts = _compact(gates)
    gvT = gv.T

    xpad = jnp.concatenate(
        [x, jnp.broadcast_to(x[:, _L - 1:, :], (_B, _TMAX - _L, _C))], axis=1)
    xflat = xpad.reshape(_B, _TMAX * _C)

    ys_flat = []
    for i in range(_NE):
        Lp, K = _LPS[i], _PS[i] * _C
        Xp = xflat[:, :_FLAT[i]].reshape(_B, Lp, K)
        y = _expert(i, rows, cnts, gvT, Xp, ve_ws[i], head_ws[i],
                    head_bs[i].reshape(1, K), jnp.asarray(_PES[i]))
        ys_flat.append(y.reshape(_B, _FLAT[i]))

    comb = _combine(rows, cnts, ys_flat)
    return comb.reshape(_B, _L, _C)


# speedup vs baseline: 1.2713x; 1.2713x over previous
import math

import numpy as np
import jax
import jax.numpy as jnp
from jax.experimental import pallas as pl

_B = 128
_L = 512
_C = 14
_D = 512
_TOPK = 5
_LAMBDA = 0.01
_FREQ = 256
_HID = _FREQ * 4

_PS = [int(p) for p in
       np.unique(np.floor(1.0 / np.fft.rfftfreq(_L)[1:]).astype(np.int64))]
_NE = len(_PS)
_EP = 48
_LPS = [int(math.ceil(_L / p)) for p in _PS]
_FLAT = [lp * p * _C for lp, p in zip(_LPS, _PS)]
_TMAX = max(lp * p for lp, p in zip(_LPS, _PS))
_EPS = np.float32(np.finfo(float).eps)
_RBR = 224
_RB = 8
_SB = 16

_n = np.arange(_L, dtype=np.float64)[:, None]
_k = np.arange(1, _FREQ + 1, dtype=np.float64)[None, :]
_ANG = 2.0 * np.pi * _n * _k / _L
_CM = np.cos(_ANG).astype(np.float32)
_SM = (-np.sin(_ANG)).astype(np.float32)

_SEL = np.zeros((16, _RBR), dtype=np.float32)
for _m in range(16):
    _SEL[_m, _m * _C:(_m + 1) * _C] = 1.0 / _C


def _pos_embed(Lp):
    position = np.arange(Lp, dtype=np.float32)[:, None]
    div_term = np.exp(np.arange(0, _D, 2, dtype=np.float32)
                      * -(math.log(10000.0) / _D))
    pe = np.zeros((Lp, _D), dtype=np.float32)
    pe[:, 0::2] = np.sin(position * div_term)
    pe[:, 1::2] = np.cos(position * div_term)
    return pe

_PES = [_pos_embed(lp) for lp in _LPS]

_HI = jax.lax.Precision.HIGHEST


def _softshrink(v):
    return jnp.where(v > _LAMBDA, v - _LAMBDA,
                     jnp.where(v < -_LAMBDA, v + _LAMBDA, 0.0))


def _router_body(xr_ref, xi_ref, w1r_ref, w1i_ref, b1r_ref, b1i_ref,
                 w2r_ref, w2i_ref, b2r_ref, b2i_ref, sel_ref, gates_ref):
    bf = jnp.bfloat16
    f32 = jnp.float32
    xr = xr_ref[...].astype(bf)
    xi = xi_ref[...].astype(bf)
    w1r = w1r_ref[...].astype(bf)
    w1i = w1i_ref[...].astype(bf)
    o1r = jnp.maximum(
        jax.lax.dot(xr, w1r, preferred_element_type=f32)
        - jax.lax.dot(xi, w1i, preferred_element_type=f32)
        + b1r_ref[...], 0.0)
    o1i = jnp.maximum(
        jax.lax.dot(xi, w1r, preferred_element_type=f32)
        + jax.lax.dot(xr, w1i, preferred_element_type=f32)
        + b1i_ref[...], 0.0)
    o1rb = o1r.astype(bf)
    o1ib = o1i.astype(bf)
    w2r = w2r_ref[...].astype(bf)
    w2i = w2i_ref[...].astype(bf)
    o2r = (jax.lax.dot(o1rb, w2r, preferred_element_type=f32)
           - jax.lax.dot(o1ib, w2i, preferred_element_type=f32)
           + b2r_ref[...])
    o2i = (jax.lax.dot(o1ib, w2r, preferred_element_type=f32)
           + jax.lax.dot(o1rb, w2i, preferred_element_type=f32)
           + b2i_ref[...])
    o2r = _softshrink(o2r)
    o2i = _softshrink(o2i)
    mag = jnp.sqrt(o2r * o2r + o2i * o2i)
    wts = jax.lax.dot(sel_ref[...], mag, precision=_HI)
    iota = jax.lax.broadcasted_iota(jnp.int32, (16, _EP), 1)
    work = jnp.where(iota < _NE, wts, -jnp.inf)
    vals, picks = [], []
    for _ in range(_TOPK):
        mx = jnp.max(work, axis=1, keepdims=True)
        cand = jnp.where(work == mx, iota, 2 * _EP)
        mn = jnp.min(cand, axis=1, keepdims=True)
        pick = iota == mn
        vals.append(mx)
        picks.append(pick)
        work = jnp.where(pick, -jnp.inf, work)
    vs = jnp.concatenate(vals, axis=1)
    vm = jnp.max(vs, axis=1, keepdims=True)
    ev = jnp.exp(vs - vm)
    g = ev / jnp.sum(ev, axis=1, keepdims=True)
    gates = jnp.zeros((16, _EP), jnp.float32)
    for kk in range(_TOPK):
        gates = gates + picks[kk].astype(jnp.float32) * g[:, kk:kk + 1]
    gates_ref[...] = gates


def _router(xr, xi, w1, b1, w2, b2):
    w2p = jnp.pad(w2, ((0, 0), (0, 0), (0, _EP - _NE)))
    b2p = jnp.pad(b2, ((0, 0), (0, _EP - _NE)))
    const = lambda s: (0, 0)
    return pl.pallas_call(
        _router_body,
        grid=(_B * _C // _RBR,),
        in_specs=[
            pl.BlockSpec((_RBR, _FREQ), lambda s: (s, 0)),
            pl.BlockSpec((_RBR, _FREQ), lambda s: (s, 0)),
            pl.BlockSpec((_FREQ, _HID), const),
            pl.BlockSpec((_FREQ, _HID), const),
            pl.BlockSpec((1, _HID), const),
            pl.BlockSpec((1, _HID), const),
            pl.BlockSpec((_HID, _EP), const),
            pl.BlockSpec((_HID, _EP), const),
            pl.BlockSpec((1, _EP), const),
            pl.BlockSpec((1, _EP), const),
            pl.BlockSpec((16, _RBR), const),
        ],
        out_specs=pl.BlockSpec((16, _EP), lambda s: (s, 0)),
        out_shape=jax.ShapeDtypeStruct((_B, _EP), jnp.float32),
    )(xr, xi,
      w1[0], w1[1], b1[0].reshape(1, -1), b1[1].reshape(1, -1),
      w2p[0], w2p[1], b2p[0].reshape(1, -1), b2p[1].reshape(1, -1),
      jnp.asarray(_SEL))


_UT = np.triu(np.ones((_B, _B), dtype=np.float32))
_ECH = 8


def _compact_body(gT_ref, ut_ref, rows_ref, gv_ref, cnt_ref):
    gT = gT_ref[...]
    m = (gT > 0.0).astype(jnp.float32)
    pos = jax.lax.dot(m, ut_ref[...], precision=_HI) - 1.0
    posi = pos.astype(jnp.int32)
    j3 = jax.lax.broadcasted_iota(jnp.int32, (_ECH, _B, _B), 2)
    oh = jnp.where((posi[:, :, None] == j3) & (m[:, :, None] > 0.0), 1.0, 0.0)
    rvec = jax.lax.broadcasted_iota(jnp.int32, (_ECH, _B, _B), 1).astype(jnp.float32)
    rows = jnp.sum(oh * rvec, axis=1)
    gv = jnp.sum(oh * gT[:, :, None], axis=1)
    cnt = jnp.sum(m, axis=1, keepdims=True)
    rows_ref[...] = rows.astype(jnp.int32)
    gv_ref[...] = gv
    cnt_ref[...] = jnp.broadcast_to(cnt, (_ECH, _B)).astype(jnp.int32)


def _compact(gates):
    gT = gates.T
    blk = lambda s: (s, 0)
    const = lambda s: (0, 0)
    return pl.pallas_call(
        _compact_body,
        grid=(_EP // _ECH,),
        in_specs=[
            pl.BlockSpec((_ECH, _B), blk),
            pl.BlockSpec((_B, _B), const),
        ],
        out_specs=[
            pl.BlockSpec((_ECH, _B), blk),
            pl.BlockSpec((_ECH, _B), blk),
            pl.BlockSpec((_ECH, _B), blk),
        ],
        out_shape=[
            jax.ShapeDtypeStruct((_EP, _B), jnp.int32),
            jax.ShapeDtypeStruct((_EP, _B), jnp.float32),
            jax.ShapeDtypeStruct((_EP, _B), jnp.int32),
        ],
    )(gT, jnp.asarray(_UT))


def _make_expert_body(i):
    Lp, K = _LPS[i], _PS[i] * _C

    def body(rows_ref, cnt_ref, gvT_ref, xp_ref, v_ref, w_ref, b_ref,
             pe_ref, o_ref):
        s = pl.program_id(0)

        @pl.when(s * _SB < cnt_ref[i, 0])
        def _():
            parts = [xp_ref[pl.ds(rows_ref[i, s * _SB + r], 1)]
                     for r in range(_SB)]
            xg = jnp.concatenate(parts, axis=0)
            xp = xg.reshape(_SB * Lp, K).astype(jnp.bfloat16)
            emb = jax.lax.dot(xp, v_ref[...].astype(jnp.bfloat16),
                              preferred_element_type=jnp.float32)
            pe = jnp.broadcast_to(pe_ref[...][None], (_SB, Lp, _D))
            emb = emb + pe.reshape(_SB * Lp, _D)
            y = jax.lax.dot(emb.astype(jnp.bfloat16),
                            w_ref[...].astype(jnp.bfloat16),
                            preferred_element_type=jnp.float32) + b_ref[...]
            gcol = gvT_ref[...][:, i:i + 1]
            o_ref[...] = jnp.exp(y.reshape(_SB, Lp, K)) * gcol[:, :, None]
    return body


def _expert(i, rows, cnts, gvT, Xp, v, w, b, pe):
    Lp, K = _LPS[i], _PS[i] * _C
    const = lambda s, rows, cnt: (0, 0)

    def omap(s, rows_sm, cnt_sm):
        nb = jnp.maximum((cnt_sm[i, 0] + _SB - 1) // _SB - 1, 0)
        return (jnp.minimum(s, nb), 0, 0)

    from jax.experimental.pallas import tpu as pltpu
    grid_spec = pltpu.PrefetchScalarGridSpec(
        num_scalar_prefetch=2,
        grid=(_B // _SB,),
        in_specs=[
            pl.BlockSpec((_SB, _EP), lambda s, rows, cnt: (s, 0)),
            pl.BlockSpec((_B, Lp, K), lambda s, rows, cnt: (0, 0, 0)),
            pl.BlockSpec((K, _D), const),
            pl.BlockSpec((_D, K), const),
            pl.BlockSpec((1, K), const),
            pl.BlockSpec((Lp, _D), const),
        ],
        out_specs=pl.BlockSpec((_SB, Lp, K), omap),
    )
    return pl.pallas_call(
        _make_expert_body(i),
        grid_spec=grid_spec,
        out_shape=jax.ShapeDtypeStruct((_B, Lp, K), jnp.float32),
    )(rows, cnts, gvT, Xp, v, w, b, pe)


def _combine_body(rows_ref, cnt_ref, *rest):
    yrefs = rest[:_NE]
    o_ref = rest[_NE]
    acc_ref = rest[_NE + 1]
    i = pl.program_id(0)
    s = pl.program_id(1)

    @pl.when((i == 0) & (s == 0))
    def _():
        acc_ref[...] = jnp.zeros((_B, _L * _C), jnp.float32)

    cnt = cnt_ref[i, 0]

    @pl.when(s * _SB < cnt)
    def _():
        yv = jax.lax.switch(
            i, [lambda j=j: yrefs[j][...][:, :_L * _C] for j in range(_NE)])
        rr = jax.lax.broadcasted_iota(jnp.int32, (_SB, 1), 0)
        valid = (s * _SB + rr) < cnt
        contrib = jnp.where(valid, yv, 0.0)
        for r in range(_SB):
            row = rows_ref[i, s * _SB + r]
            acc_ref[pl.ds(row, 1), :] += contrib[r:r + 1, :]

    @pl.when((i == _NE - 1) & (s == (_B // _SB) - 1))
    def _():
        a = acc_ref[...]
        o_ref[...] = jnp.log(jnp.where(a == 0.0, _EPS, a))


def _combine(rows, cnts, ys_flat):
    from jax.experimental.pallas import tpu as pltpu

    in_specs = []
    for j in range(_NE):
        def ymap(i, s, rows_sm, cnt_sm, j=j):
            nbj = jnp.maximum((cnt_sm[j, 0] + _SB - 1) // _SB - 1, 0)
            return (jnp.where(i < j, 0,
                              jnp.where(i > j, nbj, jnp.minimum(s, nbj))), 0)
        in_specs.append(pl.BlockSpec((_SB, _FLAT[j]), ymap))

    grid_spec = pltpu.PrefetchScalarGridSpec(
        num_scalar_prefetch=2,
        grid=(_NE, _B // _SB),
        in_specs=in_specs,
        out_specs=pl.BlockSpec((_B, _L * _C), lambda i, s, r, c: (0, 0)),
        scratch_shapes=[pltpu.VMEM((_B, _L * _C), jnp.float32)],
    )
    return pl.pallas_call(
        _combine_body,
        grid_spec=grid_spec,
        out_shape=jax.ShapeDtypeStruct((_B, _L * _C), jnp.float32),
    )(rows, cnts, *ys_flat)


def kernel(x, afno_params, ve_ws, head_ws, head_bs):
    w1, b1, w2, b2 = afno_params
    xf = jnp.fft.rfft(jnp.transpose(x, (0, 2, 1)), axis=-1)
    xr = jnp.real(xf)[:, :, 1:].reshape(_B * _C, _FREQ)
    xi = jnp.imag(xf)[:, :, 1:].reshape(_B * _C, _FREQ)
    gates = _router(xr, xi, w1, b1, w2, b2)
    rows, gv, cnts = _compact(gates)
    gvT = gv.T

    xpad = jnp.concatenate(
        [x, jnp.broadcast_to(x[:, _L - 1:, :], (_B, _TMAX - _L, _C))], axis=1)
    xflat = xpad.reshape(_B, _TMAX * _C)

    ys_flat = []
    for i in range(_NE):
        Lp, K = _LPS[i], _PS[i] * _C
        Xp = xflat[:, :_FLAT[i]].reshape(_B, Lp, K)
        y = _expert(i, rows, cnts, gvT, Xp, ve_ws[i], head_ws[i],
                    head_bs[i].reshape(1, K), jnp.asarray(_PES[i]))
        ys_flat.append(y.reshape(_B, _FLAT[i]))

    comb = _combine(rows, cnts, ys_flat)
    return comb.reshape(_B, _L, _C)


# speedup vs baseline: 1.3053x; 1.0268x over previous
import math

import numpy as np
import jax
import jax.numpy as jnp
from jax.experimental import pallas as pl

_B = 128
_L = 512
_C = 14
_D = 512
_TOPK = 5
_LAMBDA = 0.01
_FREQ = 256
_HID = _FREQ * 4

_PS = [int(p) for p in
       np.unique(np.floor(1.0 / np.fft.rfftfreq(_L)[1:]).astype(np.int64))]
_NE = len(_PS)
_EP = 48
_LPS = [int(math.ceil(_L / p)) for p in _PS]
_FLAT = [lp * p * _C for lp, p in zip(_LPS, _PS)]
_TMAX = max(lp * p for lp, p in zip(_LPS, _PS))
_EPS = np.float32(np.finfo(float).eps)
_RBR = 224
_RB = 8
_SB = 16

_SEL = np.zeros((16, _RBR), dtype=np.float32)
for _m in range(16):
    _SEL[_m, _m * _C:(_m + 1) * _C] = 1.0 / _C


def _pos_embed(Lp):
    position = np.arange(Lp, dtype=np.float32)[:, None]
    div_term = np.exp(np.arange(0, _D, 2, dtype=np.float32)
                      * -(math.log(10000.0) / _D))
    pe = np.zeros((Lp, _D), dtype=np.float32)
    pe[:, 0::2] = np.sin(position * div_term)
    pe[:, 1::2] = np.cos(position * div_term)
    return pe

_PES = [_pos_embed(lp) for lp in _LPS]

_HI = jax.lax.Precision.HIGHEST


def _softshrink(v):
    return jnp.where(v > _LAMBDA, v - _LAMBDA,
                     jnp.where(v < -_LAMBDA, v + _LAMBDA, 0.0))


def _router_body(xr_ref, xi_ref, w1r_ref, w1i_ref, b1r_ref, b1i_ref,
                 w2r_ref, w2i_ref, b2r_ref, b2i_ref, sel_ref, gates_ref):
    bf = jnp.bfloat16
    f32 = jnp.float32
    xr = xr_ref[...].astype(bf)
    xi = xi_ref[...].astype(bf)
    w1r = w1r_ref[...].astype(bf)
    w1i = w1i_ref[...].astype(bf)
    o1r = jnp.maximum(
        jax.lax.dot(xr, w1r, preferred_element_type=f32)
        - jax.lax.dot(xi, w1i, preferred_element_type=f32)
        + b1r_ref[...], 0.0)
    o1i = jnp.maximum(
        jax.lax.dot(xi, w1r, preferred_element_type=f32)
        + jax.lax.dot(xr, w1i, preferred_element_type=f32)
        + b1i_ref[...], 0.0)
    o1rb = o1r.astype(bf)
    o1ib = o1i.astype(bf)
    w2r = w2r_ref[...].astype(bf)
    w2i = w2i_ref[...].astype(bf)
    o2r = (jax.lax.dot(o1rb, w2r, preferred_element_type=f32)
           - jax.lax.dot(o1ib, w2i, preferred_element_type=f32)
           + b2r_ref[...])
    o2i = (jax.lax.dot(o1ib, w2r, preferred_element_type=f32)
           + jax.lax.dot(o1rb, w2i, preferred_element_type=f32)
           + b2i_ref[...])
    o2r = _softshrink(o2r)
    o2i = _softshrink(o2i)
    mag = jnp.sqrt(o2r * o2r + o2i * o2i)
    wts = jax.lax.dot(sel_ref[...], mag, precision=_HI)
    iota = jax.lax.broadcasted_iota(jnp.int32, (16, _EP), 1)
    work = jnp.where(iota < _NE, wts, -jnp.inf)
    vals, picks = [], []
    for _ in range(_TOPK):
        mx = jnp.max(work, axis=1, keepdims=True)
        cand = jnp.where(work == mx, iota, 2 * _EP)
        mn = jnp.min(cand, axis=1, keepdims=True)
        pick = iota == mn
        vals.append(mx)
        picks.append(pick)
        work = jnp.where(pick, -jnp.inf, work)
    vs = jnp.concatenate(vals, axis=1)
    vm = jnp.max(vs, axis=1, keepdims=True)
    ev = jnp.exp(vs - vm)
    g = ev / jnp.sum(ev, axis=1, keepdims=True)
    gates = jnp.zeros((16, _EP), jnp.float32)
    for kk in range(_TOPK):
        gates = gates + picks[kk].astype(jnp.float32) * g[:, kk:kk + 1]
    gates_ref[...] = gates


def _router(xr, xi, w1, b1, w2, b2):
    w2p = jnp.pad(w2, ((0, 0), (0, 0), (0, _EP - _NE)))
    b2p = jnp.pad(b2, ((0, 0), (0, _EP - _NE)))
    const = lambda s: (0, 0)
    return pl.pallas_call(
        _router_body,
        grid=(_B * _C // _RBR,),
        in_specs=[
            pl.BlockSpec((_RBR, _FREQ), lambda s: (s, 0)),
            pl.BlockSpec((_RBR, _FREQ), lambda s: (s, 0)),
            pl.BlockSpec((_FREQ, _HID), const),
            pl.BlockSpec((_FREQ, _HID), const),
            pl.BlockSpec((1, _HID), const),
            pl.BlockSpec((1, _HID), const),
            pl.BlockSpec((_HID, _EP), const),
            pl.BlockSpec((_HID, _EP), const),
            pl.BlockSpec((1, _EP), const),
            pl.BlockSpec((1, _EP), const),
            pl.BlockSpec((16, _RBR), const),
        ],
        out_specs=pl.BlockSpec((16, _EP), lambda s: (s, 0)),
        out_shape=jax.ShapeDtypeStruct((_B, _EP), jnp.float32),
    )(xr, xi,
      w1[0], w1[1], b1[0].reshape(1, -1), b1[1].reshape(1, -1),
      w2p[0], w2p[1], b2p[0].reshape(1, -1), b2p[1].reshape(1, -1),
      jnp.asarray(_SEL))


_UT = np.triu(np.ones((_B, _B), dtype=np.float32))
_ECH = 8


def _compact_body(gT_ref, ut_ref, rows_ref, gv_ref, cnt_ref):
    gT = gT_ref[...]
    m = (gT > 0.0).astype(jnp.float32)
    pos = jax.lax.dot(m, ut_ref[...], precision=_HI) - 1.0
    posi = pos.astype(jnp.int32)
    j3 = jax.lax.broadcasted_iota(jnp.int32, (_ECH, _B, _B), 2)
    oh = jnp.where((posi[:, :, None] == j3) & (m[:, :, None] > 0.0), 1.0, 0.0)
    rvec = jax.lax.broadcasted_iota(jnp.int32, (_ECH, _B, _B), 1).astype(jnp.float32)
    rows = jnp.sum(oh * rvec, axis=1)
    gv = jnp.sum(oh * gT[:, :, None], axis=1)
    cnt = jnp.sum(m, axis=1, keepdims=True)
    rows_ref[...] = rows.astype(jnp.int32)
    gv_ref[...] = gv
    cnt_ref[...] = jnp.broadcast_to(cnt, (_ECH, _B)).astype(jnp.int32)


def _compact(gates):
    gT = gates.T
    blk = lambda s: (s, 0)
    const = lambda s: (0, 0)
    return pl.pallas_call(
        _compact_body,
        grid=(_EP // _ECH,),
        in_specs=[
            pl.BlockSpec((_ECH, _B), blk),
            pl.BlockSpec((_B, _B), const),
        ],
        out_specs=[
            pl.BlockSpec((_ECH, _B), blk),
            pl.BlockSpec((_ECH, _B), blk),
            pl.BlockSpec((_ECH, _B), blk),
        ],
        out_shape=[
            jax.ShapeDtypeStruct((_EP, _B), jnp.int32),
            jax.ShapeDtypeStruct((_EP, _B), jnp.float32),
            jax.ShapeDtypeStruct((_EP, _B), jnp.int32),
        ],
    )(gT, jnp.asarray(_UT))


def _make_expert_body(i):
    Lp, K = _LPS[i], _PS[i] * _C

    def body(rows_ref, cnt_ref, gvT_ref, xp_ref, v_ref, w_ref, b_ref,
             pe_ref, o_ref):
        s = pl.program_id(0)

        @pl.when(s * _SB < cnt_ref[i, 0])
        def _():
            parts = [xp_ref[pl.ds(rows_ref[i, s * _SB + r], 1)]
                     for r in range(_SB)]
            xg = jnp.concatenate(parts, axis=0)
            xp = xg.reshape(_SB * Lp, K).astype(jnp.bfloat16)
            emb = jax.lax.dot(xp, v_ref[...].astype(jnp.bfloat16),
                              preferred_element_type=jnp.float32)
            pe = jnp.broadcast_to(pe_ref[...][None], (_SB, Lp, _D))
            emb = emb + pe.reshape(_SB * Lp, _D)
            y = jax.lax.dot(emb.astype(jnp.bfloat16),
                            w_ref[...].astype(jnp.bfloat16),
                            preferred_element_type=jnp.float32) + b_ref[...]
            gcol = gvT_ref[...][:, i:i + 1]
            o_ref[...] = jnp.exp(y.reshape(_SB, Lp, K)) * gcol[:, :, None]
    return body


def _expert(i, rows, cnts, gvT, Xp, v, w, b, pe):
    Lp, K = _LPS[i], _PS[i] * _C
    const = lambda s, rows, cnt: (0, 0)

    def omap(s, rows_sm, cnt_sm):
        nb = jnp.maximum((cnt_sm[i, 0] + _SB - 1) // _SB - 1, 0)
        return (jnp.minimum(s, nb), 0, 0)

    def gmap(s, rows_sm, cnt_sm):
        nb = jnp.maximum((cnt_sm[i, 0] + _SB - 1) // _SB - 1, 0)
        return (jnp.minimum(s, nb), 0)

    from jax.experimental.pallas import tpu as pltpu
    grid_spec = pltpu.PrefetchScalarGridSpec(
        num_scalar_prefetch=2,
        grid=(_B // _SB,),
        in_specs=[
            pl.BlockSpec((_SB, _EP), gmap),
            pl.BlockSpec((_B, Lp, K), lambda s, rows, cnt: (0, 0, 0)),
            pl.BlockSpec((K, _D), const),
            pl.BlockSpec((_D, K), const),
            pl.BlockSpec((1, K), const),
            pl.BlockSpec((Lp, _D), const),
        ],
        out_specs=pl.BlockSpec((_SB, Lp, K), omap),
    )
    return pl.pallas_call(
        _make_expert_body(i),
        grid_spec=grid_spec,
        out_shape=jax.ShapeDtypeStruct((_B, Lp, K), jnp.float32),
    )(rows, cnts, gvT, Xp, v, w, b, pe)


def _combine_body(rows_ref, cnt_ref, *rest):
    yrefs = rest[:_NE]
    o_ref = rest[_NE]
    acc_ref = rest[_NE + 1]
    i = pl.program_id(0)
    s = pl.program_id(1)

    @pl.when((i == 0) & (s == 0))
    def _():
        acc_ref[...] = jnp.zeros((_B, _L * _C), jnp.float32)

    cnt = cnt_ref[i, 0]

    @pl.when(s * _SB < cnt)
    def _():
        yv = jax.lax.switch(
            i, [lambda j=j: yrefs[j][...][:, :_L * _C] for j in range(_NE)])
        rr = jax.lax.broadcasted_iota(jnp.int32, (_SB, 1), 0)
        valid = (s * _SB + rr) < cnt
        contrib = jnp.where(valid, yv, 0.0)
        for r in range(_SB):
            row = rows_ref[i, s * _SB + r]
            acc_ref[pl.ds(row, 1), :] += contrib[r:r + 1, :]

    @pl.when((i == _NE - 1) & (s == (_B // _SB) - 1))
    def _():
        a = acc_ref[...]
        o_ref[...] = jnp.log(jnp.where(a == 0.0, _EPS, a))


def _combine(rows, cnts, ys_flat):
    from jax.experimental.pallas import tpu as pltpu

    in_specs = []
    for j in range(_NE):
        def ymap(i, s, rows_sm, cnt_sm, j=j):
            nbj = jnp.maximum((cnt_sm[j, 0] + _SB - 1) // _SB - 1, 0)
            return (jnp.where(i < j, 0,
                              jnp.where(i > j, nbj, jnp.minimum(s, nbj))), 0)
        in_specs.append(pl.BlockSpec((_SB, _FLAT[j]), ymap))

    grid_spec = pltpu.PrefetchScalarGridSpec(
        num_scalar_prefetch=2,
        grid=(_NE, _B // _SB),
        in_specs=in_specs,
        out_specs=pl.BlockSpec((_B, _L * _C), lambda i, s, r, c: (0, 0)),
        scratch_shapes=[pltpu.VMEM((_B, _L * _C), jnp.float32)],
    )
    return pl.pallas_call(
        _combine_body,
        grid_spec=grid_spec,
        out_shape=jax.ShapeDtypeStruct((_B, _L * _C), jnp.float32),
    )(rows, cnts, *ys_flat)


def kernel(x, afno_params, ve_ws, head_ws, head_bs):
    w1, b1, w2, b2 = afno_params
    xf = jnp.fft.rfft(jnp.transpose(x, (0, 2, 1)), axis=-1)
    xr = jnp.real(xf)[:, :, 1:].reshape(_B * _C, _FREQ)
    xi = jnp.imag(xf)[:, :, 1:].reshape(_B * _C, _FREQ)
    gates = _router(xr, xi, w1, b1, w2, b2)
    rows, gv, cnts = _compact(gates)
    gvT = gv.T

    xpad = jnp.concatenate(
        [x, jnp.broadcast_to(x[:, _L - 1:, :], (_B, _TMAX - _L, _C))], axis=1)
    xflat = xpad.reshape(_B, _TMAX * _C)

    ys_flat = []
    for i in range(_NE):
        Lp, K = _LPS[i], _PS[i] * _C
        if Lp * _PS[i] == _L:
            Xp = x.reshape(_B, Lp, K)
        else:
            Xp = xflat[:, :_FLAT[i]].reshape(_B, Lp, K)
        y = _expert(i, rows, cnts, gvT, Xp, ve_ws[i], head_ws[i],
                    head_bs[i].reshape(1, K), jnp.asarray(_PES[i]))
        ys_flat.append(y.reshape(_B, _FLAT[i]))

    comb = _combine(rows, cnts, ys_flat)
    return comb.reshape(_B, _L, _C)
